# Initial kernel scaffold; baseline (speedup 1.0000x reference)
import jax
import jax.numpy as jnp
from jax.experimental import pallas as pl


def kernel(node_emb, edge_index, rel_id, rel_emb_table, projection_table):
    raise NotImplementedError("write your pallas kernel here")


# speedup vs baseline: 2.4351x; 2.4351x over previous
import jax
import jax.numpy as jnp
from jax import lax
from jax.experimental import pallas as pl
from jax.experimental.pallas import tpu as pltpu
from jax.experimental.pallas import tpu_sc as plsc

GAMMA = 12.0
N_EDGES_TOTAL = 160000
ED = 128
RD = 16
L = 16
NC = 2
NS = 16
NW = NC * NS
G = 16
NGROUPS = N_EDGES_TOTAL // G

_GDN = lax.GatherDimensionNumbers(
    offset_dims=(), collapsed_slice_dims=(0,), start_index_map=(0,))


def _bcast_lane(vec, i):
    idx = jnp.full((L, 1), i, jnp.int32)
    return lax.gather(vec, idx, _GDN, slice_sizes=(1,),
                      mode=lax.GatherScatterMode.PROMISE_IN_BOUNDS)


def _body(node_ref, ei_ref, rid_ref, rel_ref, proj_ref, out_ref,
          hidx, tidx, ridx, head_v, tail_v, proj_v, rel_tab, score_m,
          out_v, sem):
    wid = lax.axis_index("s") * NC + lax.axis_index("c")
    ngroups = (NGROUPS - wid + NW - 1) // NW
    pltpu.sync_copy(rel_ref, rel_tab)
    lanes = lax.iota(jnp.int32, L)

    def group(t, carry):
        g = wid + t * NW
        base = g * G
        pltpu.sync_copy(ei_ref.at[0, pl.ds(base, G)], hidx)
        pltpu.sync_copy(ei_ref.at[1, pl.ds(base, G)], tidx)
        pltpu.sync_copy(rid_ref.at[pl.ds(base, G)], ridx)
        c1 = pltpu.async_copy(node_ref.at[hidx], head_v, sem)
        c2 = pltpu.async_copy(node_ref.at[tidx], tail_v, sem)
        c3 = pltpu.async_copy(proj_ref.at[ridx], proj_v, sem)
        c1.wait()
        c2.wait()
        c3.wait()
        rvec = ridx[...]

        def edge(e, carry2):
            rb = _bcast_lane(rvec, e)
            acc0 = plsc.load_gather(rel_tab, [rb * RD + lanes])
            acc1 = jnp.zeros((L,), jnp.float32)
            for c in range(ED // L):
                dv = head_v[e, pl.ds(c * L, L)] - tail_v[e, pl.ds(c * L, L)]
                for i in range(L):
                    pv = proj_v[e, pl.ds((c * L + i) * RD, RD)]
                    db = _bcast_lane(dv, i)
                    if (i % 2) == 0:
                        acc0 = acc0 + db * pv
                    else:
                        acc1 = acc1 + db * pv
            score_m[pl.ds(e * L, L)] = jnp.full(
                (L,), GAMMA / L, jnp.float32) - jnp.abs(acc0 + acc1)
            return carry2

        lax.fori_loop(0, G, edge, 0, unroll=False)

        rows = lanes * L
        sv = plsc.load_gather(score_m, [rows])
        for j in range(1, L):
            sv = sv + plsc.load_gather(
                score_m, [rows + jnp.full((L,), j, jnp.int32)])
        out_v[...] = sv
        pltpu.sync_copy(out_v, out_ref.at[pl.ds(base, G)])
        return carry

    lax.fori_loop(0, ngroups, group, 0, unroll=False)


@jax.jit
def _sc_call(node_emb, edge_index, rel_id, rel_emb_table, projection_table):
    mesh = plsc.VectorSubcoreMesh(core_axis_name="c", subcore_axis_name="s")
    f = pl.kernel(
        _body,
        out_type=jax.ShapeDtypeStruct((N_EDGES_TOTAL,), jnp.float32),
        mesh=mesh,
        scratch_types=[
            pltpu.VMEM((G,), jnp.int32),
            pltpu.VMEM((G,), jnp.int32),
            pltpu.VMEM((G,), jnp.int32),
            pltpu.VMEM((G, ED), jnp.float32),
            pltpu.VMEM((G, ED), jnp.float32),
            pltpu.VMEM((G, ED * RD), jnp.float32),
            pltpu.VMEM((1000 * RD,), jnp.float32),
            pltpu.VMEM((G * L,), jnp.float32),
            pltpu.VMEM((G,), jnp.float32),
            pltpu.SemaphoreType.DMA,
        ],
        compiler_params=pltpu.CompilerParams(needs_layout_passes=False),
    )
    return f(node_emb, edge_index, rel_id, rel_emb_table.reshape(-1),
             projection_table)


def kernel(node_emb, edge_index, rel_id, rel_emb_table, projection_table):
    return _sc_call(node_emb, edge_index, rel_id, rel_emb_table,
                    projection_table)


# speedup vs baseline: 3.7837x; 1.5538x over previous
import jax
import jax.numpy as jnp
from jax import lax
from jax.experimental import pallas as pl
from jax.experimental.pallas import tpu as pltpu
from jax.experimental.pallas import tpu_sc as plsc

GAMMA = 12.0
N_EDGES_TOTAL = 160000
ED = 128
RD = 16
L = 16
NC = 2
NS = 16
NW = NC * NS
G = 16
NGROUPS = N_EDGES_TOTAL // G
TMAX = (NGROUPS + NW - 1) // NW

_GDN = lax.GatherDimensionNumbers(
    offset_dims=(), collapsed_slice_dims=(0,), start_index_map=(0,))


def _bcast_lane(vec, i):
    idx = jnp.full((L, 1), i, jnp.int32)
    return lax.gather(vec, idx, _GDN, slice_sizes=(1,),
                      mode=lax.GatherScatterMode.PROMISE_IN_BOUNDS)


def _body(node_ref, ei_ref, rid_ref, rel_ref, proj_ref, out_ref,
          hidx0, tidx0, ridx0, head0, tail0, proj0, sem0,
          hidx1, tidx1, ridx1, head1, tail1, proj1, sem1,
          rel_tab, score_m, out_v):
    wid = lax.axis_index("s") * NC + lax.axis_index("c")
    ngroups = (NGROUPS - wid + NW - 1) // NW
    pltpu.sync_copy(rel_ref, rel_tab)
    lanes = lax.iota(jnp.int32, L)

    bufs = ((hidx0, tidx0, ridx0, head0, tail0, proj0, sem0),
            (hidx1, tidx1, ridx1, head1, tail1, proj1, sem1))

    def issue(t, buf):
        hidx, tidx, ridx, head_v, tail_v, proj_v, sem = buf

        @pl.when(t < ngroups)
        def _():
            base = (wid + t * NW) * G
            pltpu.sync_copy(ei_ref.at[0, pl.ds(base, G)], hidx)
            pltpu.sync_copy(ei_ref.at[1, pl.ds(base, G)], tidx)
            pltpu.sync_copy(rid_ref.at[pl.ds(base, G)], ridx)
            pltpu.async_copy(node_ref.at[hidx], head_v, sem)
            pltpu.async_copy(node_ref.at[tidx], tail_v, sem)
            pltpu.async_copy(proj_ref.at[ridx], proj_v, sem)

    def compute(t, buf):
        hidx, tidx, ridx, head_v, tail_v, proj_v, sem = buf

        @pl.when(t < ngroups)
        def _():
            base = (wid + t * NW) * G
            pltpu.make_async_copy(node_ref.at[hidx], head_v, sem).wait()
            pltpu.make_async_copy(node_ref.at[tidx], tail_v, sem).wait()
            pltpu.make_async_copy(proj_ref.at[ridx], proj_v, sem).wait()
            rvec = ridx[...]

            def edge(e, carry2):
                rb = _bcast_lane(rvec, e)
                acc0 = plsc.load_gather(rel_tab, [rb * RD + lanes])
                acc1 = jnp.zeros((L,), jnp.float32)
                for c in range(ED // L):
                    dv = (head_v[e, pl.ds(c * L, L)]
                          - tail_v[e, pl.ds(c * L, L)])
                    for i in range(L):
                        pv = proj_v[e, pl.ds((c * L + i) * RD, RD)]
                        db = _bcast_lane(dv, i)
                        if (i % 2) == 0:
                            acc0 = acc0 + db * pv
                        else:
                            acc1 = acc1 + db * pv
                score_m[pl.ds(e * L, L)] = jnp.full(
                    (L,), GAMMA / L, jnp.float32) - jnp.abs(acc0 + acc1)
                return carry2

            lax.fori_loop(0, G, edge, 0, unroll=False)

            rows = lanes * L
            sv = plsc.load_gather(score_m, [rows])
            for j in range(1, L):
                sv = sv + plsc.load_gather(
                    score_m, [rows + jnp.full((L,), j, jnp.int32)])
            out_v[...] = sv
            pltpu.sync_copy(out_v, out_ref.at[pl.ds(base, G)])

    issue(0, bufs[0])

    def pair(p, carry):
        t = p * 2
        issue(t + 1, bufs[1])
        compute(t, bufs[0])
        issue(t + 2, bufs[0])
        compute(t + 1, bufs[1])
        return carry

    lax.fori_loop(0, (TMAX + 1) // 2, pair, 0, unroll=False)


@jax.jit
def _sc_call(node_emb, edge_index, rel_id, rel_emb_table, projection_table):
    mesh = plsc.VectorSubcoreMesh(core_axis_name="c", subcore_axis_name="s")
    staging = [
        pltpu.VMEM((G,), jnp.int32),
        pltpu.VMEM((G,), jnp.int32),
        pltpu.VMEM((G,), jnp.int32),
        pltpu.VMEM((G, ED), jnp.float32),
        pltpu.VMEM((G, ED), jnp.float32),
        pltpu.VMEM((G, ED * RD), jnp.float32),
        pltpu.SemaphoreType.DMA,
    ]
    f = pl.kernel(
        _body,
        out_type=jax.ShapeDtypeStruct((N_EDGES_TOTAL,), jnp.float32),
        mesh=mesh,
        scratch_types=staging + staging + [
            pltpu.VMEM((1000 * RD,), jnp.float32),
            pltpu.VMEM((G * L,), jnp.float32),
            pltpu.VMEM((G,), jnp.float32),
        ],
        compiler_params=pltpu.CompilerParams(needs_layout_passes=False),
    )
    return f(node_emb, edge_index, rel_id, rel_emb_table.reshape(-1),
             projection_table)


def kernel(node_emb, edge_index, rel_id, rel_emb_table, projection_table):
    return _sc_call(node_emb, edge_index, rel_id, rel_emb_table,
                    projection_table)


# speedup vs baseline: 4.0209x; 1.0627x over previous
import jax
import jax.numpy as jnp
from jax import lax
from jax.experimental import pallas as pl
from jax.experimental.pallas import tpu as pltpu
from jax.experimental.pallas import tpu_sc as plsc

GAMMA = 12.0
N_EDGES_TOTAL = 160000
ED = 128
RD = 16
L = 16
NC = 2
NS = 16
NW = NC * NS
G = 16
NGROUPS = N_EDGES_TOTAL // G
TMAX = (NGROUPS + NW - 1) // NW

_GDN = lax.GatherDimensionNumbers(
    offset_dims=(), collapsed_slice_dims=(0,), start_index_map=(0,))


def _bcast_lane(vec, i):
    idx = jnp.full((L, 1), i, jnp.int32)
    return lax.gather(vec, idx, _GDN, slice_sizes=(1,),
                      mode=lax.GatherScatterMode.PROMISE_IN_BOUNDS)


def _body(node_ref, ei_ref, rid_ref, rel_ref, proj_ref, out_ref,
          hidx0, tidx0, ridx0, head0, tail0, proj0, sem0,
          hidx1, tidx1, ridx1, head1, tail1, proj1, sem1,
          rel_tab, score_m, out_v):
    wid = lax.axis_index("s") * NC + lax.axis_index("c")
    ngroups = (NGROUPS - wid + NW - 1) // NW
    pltpu.sync_copy(rel_ref, rel_tab)
    lanes = lax.iota(jnp.int32, L)

    bufs = ((hidx0, tidx0, ridx0, head0, tail0, proj0, sem0),
            (hidx1, tidx1, ridx1, head1, tail1, proj1, sem1))

    def issue(t, buf):
        hidx, tidx, ridx, head_v, tail_v, proj_v, sem = buf

        @pl.when(t < ngroups)
        def _():
            base = (wid + t * NW) * G
            pltpu.sync_copy(ei_ref.at[0, pl.ds(base, G)], hidx)
            pltpu.sync_copy(ei_ref.at[1, pl.ds(base, G)], tidx)
            pltpu.sync_copy(rid_ref.at[pl.ds(base, G)], ridx)
            pltpu.async_copy(node_ref.at[hidx], head_v, sem)
            pltpu.async_copy(node_ref.at[tidx], tail_v, sem)
            pltpu.async_copy(proj_ref.at[ridx], proj_v, sem)

    def compute(t, buf):
        hidx, tidx, ridx, head_v, tail_v, proj_v, sem = buf

        @pl.when(t < ngroups)
        def _():
            base = (wid + t * NW) * G
            pltpu.make_async_copy(node_ref.at[hidx], head_v, sem).wait()
            pltpu.make_async_copy(node_ref.at[tidx], tail_v, sem).wait()
            pltpu.make_async_copy(proj_ref.at[ridx], proj_v, sem).wait()
            rvec = ridx[...]

            def edge(e, carry2):
                rb = _bcast_lane(rvec, e)
                acc0 = plsc.load_gather(rel_tab, [rb * RD + lanes])
                acc1 = jnp.zeros((L,), jnp.float32)
                for c in range(ED // L):
                    dv = (head_v[e, pl.ds(c * L, L)]
                          - tail_v[e, pl.ds(c * L, L)])
                    for m in range(L // 2):
                        pw = proj_v[e, pl.ds((c * (L // 2) + m) * RD, RD)]
                        pa, pb = plsc.unpack(
                            plsc.bitcast(pw, jnp.bfloat16),
                            format=plsc.PackFormat.INTERLEAVED)
                        acc0 = acc0 + _bcast_lane(dv, 2 * m) * pa
                        acc1 = acc1 + _bcast_lane(dv, 2 * m + 1) * pb
                score_m[pl.ds(e * L, L)] = jnp.full(
                    (L,), GAMMA / L, jnp.float32) - jnp.abs(acc0 + acc1)
                return carry2

            lax.fori_loop(0, G, edge, 0, unroll=False)

            rows = lanes * L
            sv = plsc.load_gather(score_m, [rows])
            for j in range(1, L):
                sv = sv + plsc.load_gather(
                    score_m, [rows + jnp.full((L,), j, jnp.int32)])
            out_v[...] = sv
            pltpu.sync_copy(out_v, out_ref.at[pl.ds(base, G)])

    issue(0, bufs[0])

    def pair(p, carry):
        t = p * 2
        issue(t + 1, bufs[1])
        compute(t, bufs[0])
        issue(t + 2, bufs[0])
        compute(t + 1, bufs[1])
        return carry

    lax.fori_loop(0, (TMAX + 1) // 2, pair, 0, unroll=False)


@jax.jit
def _sc_call(node_emb, edge_index, rel_id, rel_emb_table, projection_table):
    mesh = plsc.VectorSubcoreMesh(core_axis_name="c", subcore_axis_name="s")
    staging = [
        pltpu.VMEM((G,), jnp.int32),
        pltpu.VMEM((G,), jnp.int32),
        pltpu.VMEM((G,), jnp.int32),
        pltpu.VMEM((G, ED), jnp.float32),
        pltpu.VMEM((G, ED), jnp.float32),
        pltpu.VMEM((G, ED * RD // 2), jnp.int32),
        pltpu.SemaphoreType.DMA,
    ]
    f = pl.kernel(
        _body,
        out_type=jax.ShapeDtypeStruct((N_EDGES_TOTAL,), jnp.float32),
        mesh=mesh,
        scratch_types=staging + staging + [
            pltpu.VMEM((1000 * RD,), jnp.float32),
            pltpu.VMEM((G * L,), jnp.float32),
            pltpu.VMEM((G,), jnp.float32),
        ],
        compiler_params=pltpu.CompilerParams(needs_layout_passes=False),
    )
    proj_bf = (projection_table.reshape(-1, ED // 2, 2, RD)
               .swapaxes(2, 3).astype(jnp.bfloat16)
               .reshape(-1, ED * RD // 2, 2))
    proj_i32 = lax.bitcast_convert_type(proj_bf, jnp.int32)
    return f(node_emb, edge_index, rel_id, rel_emb_table.reshape(-1),
             proj_i32)


def kernel(node_emb, edge_index, rel_id, rel_emb_table, projection_table):
    return _sc_call(node_emb, edge_index, rel_id, rel_emb_table,
                    projection_table)


# speedup vs baseline: 4.2149x; 1.0482x over previous
import jax
import jax.numpy as jnp
from jax import lax
from jax.experimental import pallas as pl
from jax.experimental.pallas import tpu as pltpu
from jax.experimental.pallas import tpu_sc as plsc

GAMMA = 12.0
N_EDGES_TOTAL = 160000
ED = 128
RD = 16
L = 16
NC = 2
NS = 16
NW = NC * NS
G = 16
NGROUPS = N_EDGES_TOTAL // G
TMAX = (NGROUPS + NW - 1) // NW

_GDN = lax.GatherDimensionNumbers(
    offset_dims=(), collapsed_slice_dims=(0,), start_index_map=(0,))


def _bcast_lane(vec, i):
    idx = jnp.full((L, 1), i, jnp.int32)
    return lax.gather(vec, idx, _GDN, slice_sizes=(1,),
                      mode=lax.GatherScatterMode.PROMISE_IN_BOUNDS)


def _body(node_ref, ei_ref, rid_ref, rel_ref, proj_ref, out_ref,
          hidx0, tidx0, ridx0, head0, tail0, proj0, sem0,
          hidx1, tidx1, ridx1, head1, tail1, proj1, sem1,
          rel_tab, score_m, out_v):
    wid = lax.axis_index("s") * NC + lax.axis_index("c")
    ngroups = (NGROUPS - wid + NW - 1) // NW
    pltpu.sync_copy(rel_ref, rel_tab)
    lanes = lax.iota(jnp.int32, L)

    bufs = ((hidx0, tidx0, ridx0, head0, tail0, proj0, sem0),
            (hidx1, tidx1, ridx1, head1, tail1, proj1, sem1))

    def issue(t, buf):
        hidx, tidx, ridx, head_v, tail_v, proj_v, sem = buf

        @pl.when(t < ngroups)
        def _():
            base = (wid + t * NW) * G
            pltpu.sync_copy(ei_ref.at[0, pl.ds(base, G)], hidx)
            pltpu.sync_copy(ei_ref.at[1, pl.ds(base, G)], tidx)
            pltpu.sync_copy(rid_ref.at[pl.ds(base, G)], ridx)
            pltpu.async_copy(node_ref.at[hidx], head_v, sem)
            pltpu.async_copy(node_ref.at[tidx], tail_v, sem)
            pltpu.async_copy(proj_ref.at[ridx], proj_v, sem)

    def compute(t, buf):
        hidx, tidx, ridx, head_v, tail_v, proj_v, sem = buf

        @pl.when(t < ngroups)
        def _():
            base = (wid + t * NW) * G
            pltpu.make_async_copy(node_ref.at[hidx], head_v, sem).wait()
            pltpu.make_async_copy(node_ref.at[tidx], tail_v, sem).wait()
            pltpu.make_async_copy(proj_ref.at[ridx], proj_v, sem).wait()
            rvec = ridx[...]

            @plsc.parallel_loop(0, G, 1, unroll=2)
            def edge(e):
                rb = _bcast_lane(rvec, e)
                acc0 = plsc.load_gather(rel_tab, [rb * RD + lanes])
                acc1 = jnp.zeros((L,), jnp.float32)
                for c in range(ED // L):
                    dv = (head_v[e, pl.ds(c * L, L)]
                          - tail_v[e, pl.ds(c * L, L)])
                    for m in range(L // 2):
                        pw = proj_v[e, pl.ds((c * (L // 2) + m) * RD, RD)]
                        pa, pb = plsc.unpack(
                            plsc.bitcast(pw, jnp.bfloat16),
                            format=plsc.PackFormat.INTERLEAVED)
                        acc0 = acc0 + _bcast_lane(dv, 2 * m) * pa
                        acc1 = acc1 + _bcast_lane(dv, 2 * m + 1) * pb
                score_m[pl.ds(e * L, L)] = jnp.full(
                    (L,), GAMMA / L, jnp.float32) - jnp.abs(acc0 + acc1)

            rows = lanes * L
            sv = plsc.load_gather(score_m, [rows])
            for j in range(1, L):
                sv = sv + plsc.load_gather(
                    score_m, [rows + jnp.full((L,), j, jnp.int32)])
            out_v[...] = sv
            pltpu.sync_copy(out_v, out_ref.at[pl.ds(base, G)])

    issue(0, bufs[0])

    def pair(p, carry):
        t = p * 2
        issue(t + 1, bufs[1])
        compute(t, bufs[0])
        issue(t + 2, bufs[0])
        compute(t + 1, bufs[1])
        return carry

    lax.fori_loop(0, (TMAX + 1) // 2, pair, 0, unroll=False)


@jax.jit
def _sc_call(node_emb, edge_index, rel_id, rel_emb_table, projection_table):
    mesh = plsc.VectorSubcoreMesh(core_axis_name="c", subcore_axis_name="s")
    staging = [
        pltpu.VMEM((G,), jnp.int32),
        pltpu.VMEM((G,), jnp.int32),
        pltpu.VMEM((G,), jnp.int32),
        pltpu.VMEM((G, ED), jnp.float32),
        pltpu.VMEM((G, ED), jnp.float32),
        pltpu.VMEM((G, ED * RD // 2), jnp.int32),
        pltpu.SemaphoreType.DMA,
    ]
    f = pl.kernel(
        _body,
        out_type=jax.ShapeDtypeStruct((N_EDGES_TOTAL,), jnp.float32),
        mesh=mesh,
        scratch_types=staging + staging + [
            pltpu.VMEM((1000 * RD,), jnp.float32),
            pltpu.VMEM((G * L,), jnp.float32),
            pltpu.VMEM((G,), jnp.float32),
        ],
        compiler_params=pltpu.CompilerParams(needs_layout_passes=False),
    )
    proj_bf = (projection_table.reshape(-1, ED // 2, 2, RD)
               .swapaxes(2, 3).astype(jnp.bfloat16)
               .reshape(-1, ED * RD // 2, 2))
    proj_i32 = lax.bitcast_convert_type(proj_bf, jnp.int32)
    return f(node_emb, edge_index, rel_id, rel_emb_table.reshape(-1),
             proj_i32)


def kernel(node_emb, edge_index, rel_id, rel_emb_table, projection_table):
    return _sc_call(node_emb, edge_index, rel_id, rel_emb_table,
                    projection_table)
